# baseline (speedup 1.0000x reference)
import functools

import jax
import jax.numpy as jnp
from jax import lax
from jax.experimental import pallas as pl
from jax.experimental.pallas import tpu as pltpu
from jax.experimental.pallas import tpu_sc as plsc

N = 10000
D = 128
E = 320000
NUM_CONVS = 8

NC = 2
NS = 16
NW = NC * NS
DUMMY = N

NPASS = 4
PW = 2504
ACCR = 2560
ARPT = ACCR // NS
GRPT = 632
GTR = 10112
CH = 64
NCHP = 40
SLOTS = NCHP * CH
LDUMMY = PW

EPT = E // NW
DCH = 128
DNCHUNK = 80
DEPT_PAD = DNCHUNK * DCH
DNPAD = 10240
DRPT = DNPAD // NS

_mesh = plsc.VectorSubcoreMesh(
    core_axis_name="c", subcore_axis_name="s", num_cores=NC, num_subcores=NS)


@functools.partial(
    pl.kernel,
    out_type=jax.ShapeDtypeStruct((NC, NPASS, ACCR, D), jnp.float32),
    mesh=_mesh,
    scratch_types=[
        pltpu.VMEM((NCHP, CH), jnp.int32),
        pltpu.VMEM((NCHP, CH), jnp.int32),
        pltpu.VMEM((2, CH, D), jnp.float32),
        pltpu.VMEM_SHARED((GTR, D), jnp.float32),
        pltpu.VMEM_SHARED((ACCR, D), jnp.float32),
        [pltpu.SemaphoreType.DMA] * 2,
        [pltpu.SemaphoreType.DMA] * 2,
    ],
)
def _sc_segsum(g_hbm, gsrc_hbm, dstp_hbm, zrows_hbm, out_hbm,
               gidx, sidx, rows, gtab, acc, gsems, ssems):
    c = lax.axis_index("c")
    s = lax.axis_index("s")
    wid = c * NS + s
    @pl.when(s < NS - 1)
    def _():
        pltpu.sync_copy(g_hbm.at[pl.ds(s * GRPT, GRPT)],
                        gtab.at[pl.ds(s * GRPT, GRPT)])

    @pl.when(s == NS - 1)
    def _():
        pltpu.sync_copy(g_hbm.at[pl.ds((NS - 1) * GRPT, N - (NS - 1) * GRPT)],
                        gtab.at[pl.ds((NS - 1) * GRPT, N - (NS - 1) * GRPT)])

    def gather(j, b):
        pltpu.async_copy(gtab.at[gidx.at[j]], rows.at[b], gsems[b])

    def gather_wait(j, b):
        pltpu.make_async_copy(gtab.at[gidx.at[j]], rows.at[b], gsems[b]).wait()

    def scat(j, b):
        pltpu.async_copy(rows.at[b], acc.at[sidx.at[j]], ssems[b], add=True)

    def scat_wait(j, b):
        pltpu.make_async_copy(rows.at[b], acc.at[sidx.at[j]], ssems[b]).wait()

    for p in range(NPASS):
        pltpu.sync_copy(zrows_hbm, acc.at[pl.ds(s * ARPT, ARPT)])
        pltpu.sync_copy(gsrc_hbm.at[p, wid], gidx)
        pltpu.sync_copy(dstp_hbm.at[p, wid], sidx)
        plsc.subcore_barrier()

        for b in range(2):
            gather(b, b)

        def body(i, carry):
            for b in range(2):
                j = i * 2 + b
                gather_wait(j, b)
                scat(j, b)
            for b in range(2):
                j = i * 2 + b
                scat_wait(j, b)

                @pl.when(j + 2 < NCHP)
                def _():
                    gather(j + 2, b)
            return carry

        lax.fori_loop(0, NCHP // 2, body, 0)
        plsc.subcore_barrier()
        pltpu.sync_copy(acc.at[pl.ds(s * ARPT, ARPT)],
                        out_hbm.at[c, p, pl.ds(s * ARPT, ARPT)])


@functools.partial(
    pl.kernel,
    out_type=jax.ShapeDtypeStruct((NC, DNPAD), jnp.float32),
    mesh=_mesh,
    scratch_types=[
        pltpu.VMEM((DNCHUNK, DCH), jnp.int32),
        pltpu.VMEM((DCH,), jnp.float32),
        pltpu.VMEM_SHARED((DNPAD,), jnp.float32),
        pltpu.SemaphoreType.DMA,
    ],
)
def _sc_degree(srcp_hbm, ones_hbm, z1d_hbm, out_hbm, sidx, ones, acc, sem):
    c = lax.axis_index("c")
    s = lax.axis_index("s")
    wid = c * NS + s
    pltpu.sync_copy(srcp_hbm.at[wid], sidx)
    pltpu.sync_copy(ones_hbm, ones)
    pltpu.sync_copy(z1d_hbm, acc.at[pl.ds(s * DRPT, DRPT)])
    plsc.subcore_barrier()

    def body(j, carry):
        pltpu.sync_copy(ones, acc.at[sidx.at[j]], add=True)
        return carry

    lax.fori_loop(0, DNCHUNK, body, 0)
    plsc.subcore_barrier()
    pltpu.sync_copy(acc.at[pl.ds(s * DRPT, DRPT)],
                    out_hbm.at[c, pl.ds(s * DRPT, DRPT)])


def _dinv_body(dA_ref, dB_ref, o_ref):
    deg = dA_ref[...] + dB_ref[...]
    o_ref[...] = jnp.where(deg > 0, lax.rsqrt(deg), 0.0)


def _scale_body(h_ref, dinv_ref, o_ref):
    o_ref[...] = h_ref[...] * dinv_ref[...]


def _tc_dinv(degA, degB):
    return pl.pallas_call(
        _dinv_body,
        out_shape=jax.ShapeDtypeStruct(degA.shape, jnp.float32),
    )(degA, degB)


def _tc_scale(h, dinv_col):
    return pl.pallas_call(
        _scale_body,
        out_shape=jax.ShapeDtypeStruct((N, D), jnp.float32),
    )(h, dinv_col)


def _layer_body(relu, resid, want_g, h_ref, acc_ref, dinv_ref, w0_ref, w1_ref,
                b_ref, *rest):
    if resid:
        yres_ref, out_ref, *grest = rest
    else:
        out_ref, *grest = rest
    parts = []
    for p in range(NPASS):
        parts.append(acc_ref[0, p, pl.ds(0, PW), :]
                     + acc_ref[1, p, pl.ds(0, PW), :])
    t = jnp.concatenate(parts, axis=0)[:N, :] * dinv_ref[...]
    out = (jnp.dot(h_ref[...], w0_ref[...], preferred_element_type=jnp.float32)
           - jnp.dot(t, w1_ref[...], preferred_element_type=jnp.float32)
           + b_ref[...])
    if relu:
        out = jnp.maximum(out, 0.0)
    if resid:
        out = (yres_ref[...] + out) * 0.5
    out_ref[...] = out
    if want_g:
        grest[0][...] = out * dinv_ref[...]


def _tc_layer(h, acc, dinv_col, w0, w1, bk, yres=None, relu=True, want_g=True):
    out_shape = [jax.ShapeDtypeStruct((N, D), jnp.float32)]
    if want_g:
        out_shape.append(jax.ShapeDtypeStruct((N, D), jnp.float32))
    args = [h, acc, dinv_col, w0, w1, bk]
    if yres is not None:
        args.append(yres)
    res = pl.pallas_call(
        functools.partial(_layer_body, relu, yres is not None, want_g),
        out_shape=out_shape,
    )(*args)
    return res if want_g else (res[0], None)


def kernel(x, edge_index, W0, W1, b):
    src = edge_index[0]
    dst = edge_index[1]
    keep = src != dst

    dpad = ((0, 0), (0, DEPT_PAD - EPT))
    srcp = jnp.pad(jnp.where(keep, src, DUMMY).reshape(NW, EPT), dpad,
                   constant_values=DUMMY).reshape(NW, DNCHUNK, DCH)

    pid = jnp.clip(dst // PW, 0, NPASS - 1)
    pid = jnp.where(keep, pid, NPASS - 1)
    order = jnp.argsort(pid)
    spass = src[order]
    dpass = dst[order]
    kpass = keep[order]
    counts = jnp.bincount(pid, length=NPASS)
    starts = jnp.concatenate([jnp.zeros((1,), counts.dtype),
                              jnp.cumsum(counts)[:-1]])
    rank = jnp.arange(NW * SLOTS, dtype=jnp.int32)[None, :]
    epos = jnp.clip(starts[:, None] + rank, 0, E - 1)
    valid = rank < counts[:, None]
    inpass = valid & kpass[epos] & (jnp.clip(dpass[epos] // PW, 0, NPASS - 1)
                                    == jnp.arange(NPASS)[:, None])
    gsrc = jnp.where(valid, spass[epos], 0)
    ldst = jnp.where(inpass,
                     dpass[epos] - jnp.arange(NPASS, dtype=dst.dtype)[:, None] * PW,
                     LDUMMY)
    gsrc = gsrc.reshape(NPASS, NW, NCHP, CH).astype(jnp.int32)
    ldst = ldst.reshape(NPASS, NW, NCHP, CH).astype(jnp.int32)

    zrows = jnp.zeros((ARPT, D), jnp.float32)
    z1d = jnp.zeros((DRPT,), jnp.float32)
    ones = jnp.ones((DCH,), jnp.float32)

    deg_parts = _sc_degree(srcp, ones, z1d)
    dinv2d = _tc_dinv(deg_parts[0].reshape(DNPAD // D, D),
                      deg_parts[1].reshape(DNPAD // D, D))
    dinv_col = dinv2d.reshape(DNPAD)[:N].reshape(N, 1)

    g = _tc_scale(x, dinv_col)

    def cheb(k, h, yres=None, relu=True, want_g=True):
        acc = _sc_segsum(g_holder[0], gsrc, ldst, zrows)
        return _tc_layer(h, acc, dinv_col, W0[k], W1[k],
                         b[k].reshape(1, D), yres=yres, relu=relu,
                         want_g=want_g)

    g_holder = [g]
    y, gy = cheb(0, x)
    g_holder[0] = gy
    for blk in range(3):
        k = 1 + 2 * blk
        h1, gh = cheb(k, y)
        g_holder[0] = gh
        y, gy = cheb(k + 1, h1, yres=y)
        g_holder[0] = gy
    y2, _ = cheb(7, y, relu=False, want_g=False)
    return (y2, y)


# speedup vs baseline: 1.1057x; 1.1057x over previous
import functools

import jax
import jax.numpy as jnp
from jax import lax
from jax.experimental import pallas as pl
from jax.experimental.pallas import tpu as pltpu
from jax.experimental.pallas import tpu_sc as plsc

N = 10000
D = 128
E = 320000
NUM_CONVS = 8

NC = 2
NS = 16
NW = NC * NS
DUMMY = N

NPASS = 4
PW = 2504
ACCR = 2560
ARPT = ACCR // NS
GRPT = 632
GTR = 10112
CH = 64
NCHP = 40
SLOTS = NCHP * CH
LDUMMY = PW

EPT = E // NW
DCH = 128
DNCHUNK = 80
DEPT_PAD = DNCHUNK * DCH
DNPAD = 10240
DRPT = DNPAD // NS

_mesh = plsc.VectorSubcoreMesh(
    core_axis_name="c", subcore_axis_name="s", num_cores=NC, num_subcores=NS)


@functools.partial(
    pl.kernel,
    out_type=jax.ShapeDtypeStruct((NC, NPASS, ACCR, D), jnp.float32),
    mesh=_mesh,
    scratch_types=[
        pltpu.VMEM((NCHP, CH), jnp.int32),
        pltpu.VMEM((NCHP, CH), jnp.int32),
        pltpu.VMEM((2, CH, D), jnp.float32),
        pltpu.VMEM_SHARED((GTR, D), jnp.float32),
        pltpu.VMEM_SHARED((ACCR, D), jnp.float32),
        [pltpu.SemaphoreType.DMA] * 2,
        [pltpu.SemaphoreType.DMA] * 2,
    ],
)
def _sc_segsum(g_hbm, gsrc_hbm, dstp_hbm, zrows_hbm, out_hbm,
               gidx, sidx, rows, gtab, acc, gsems, ssems):
    c = lax.axis_index("c")
    s = lax.axis_index("s")
    wid = c * NS + s
    @pl.when(s < NS - 1)
    def _():
        pltpu.sync_copy(g_hbm.at[pl.ds(s * GRPT, GRPT)],
                        gtab.at[pl.ds(s * GRPT, GRPT)])

    @pl.when(s == NS - 1)
    def _():
        pltpu.sync_copy(g_hbm.at[pl.ds((NS - 1) * GRPT, N - (NS - 1) * GRPT)],
                        gtab.at[pl.ds((NS - 1) * GRPT, N - (NS - 1) * GRPT)])

    def gather(j, b):
        pltpu.async_copy(gtab.at[gidx.at[j]], rows.at[b], gsems[b])

    def gather_wait(j, b):
        pltpu.make_async_copy(gtab.at[gidx.at[j]], rows.at[b], gsems[b]).wait()

    def scat(j, b):
        pltpu.async_copy(rows.at[b], acc.at[sidx.at[j]], ssems[b], add=True)

    def scat_wait(j, b):
        pltpu.make_async_copy(rows.at[b], acc.at[sidx.at[j]], ssems[b]).wait()

    for p in range(NPASS):
        pltpu.sync_copy(zrows_hbm, acc.at[pl.ds(s * ARPT, ARPT)])
        pltpu.sync_copy(gsrc_hbm.at[p, wid], gidx)
        pltpu.sync_copy(dstp_hbm.at[p, wid], sidx)
        plsc.subcore_barrier()

        for b in range(2):
            gather(b, b)

        def body(i, carry):
            for b in range(2):
                j = i * 2 + b
                gather_wait(j, b)
                scat(j, b)
            for b in range(2):
                j = i * 2 + b
                scat_wait(j, b)

                @pl.when(j + 2 < NCHP)
                def _():
                    gather(j + 2, b)
            return carry

        lax.fori_loop(0, NCHP // 2, body, 0)
        plsc.subcore_barrier()
        pltpu.sync_copy(acc.at[pl.ds(s * ARPT, ARPT)],
                        out_hbm.at[c, p, pl.ds(s * ARPT, ARPT)])


@functools.partial(
    pl.kernel,
    out_type=jax.ShapeDtypeStruct((NC, DNPAD), jnp.float32),
    mesh=_mesh,
    scratch_types=[
        pltpu.VMEM((DNCHUNK, DCH), jnp.int32),
        pltpu.VMEM((DCH,), jnp.float32),
        pltpu.VMEM_SHARED((DNPAD,), jnp.float32),
        pltpu.SemaphoreType.DMA,
    ],
)
def _sc_degree(srcp_hbm, ones_hbm, z1d_hbm, out_hbm, sidx, ones, acc, sem):
    c = lax.axis_index("c")
    s = lax.axis_index("s")
    wid = c * NS + s
    pltpu.sync_copy(srcp_hbm.at[wid], sidx)
    pltpu.sync_copy(ones_hbm, ones)
    pltpu.sync_copy(z1d_hbm, acc.at[pl.ds(s * DRPT, DRPT)])
    plsc.subcore_barrier()

    def body(j, carry):
        pltpu.sync_copy(ones, acc.at[sidx.at[j]], add=True)
        return carry

    lax.fori_loop(0, DNCHUNK, body, 0)
    plsc.subcore_barrier()
    pltpu.sync_copy(acc.at[pl.ds(s * DRPT, DRPT)],
                    out_hbm.at[c, pl.ds(s * DRPT, DRPT)])


def _dinv_body(dA_ref, dB_ref, o_ref):
    deg = dA_ref[...] + dB_ref[...]
    o_ref[...] = jnp.where(deg > 0, lax.rsqrt(deg), 0.0)


def _scale_body(h_ref, dinv_ref, o_ref):
    o_ref[...] = h_ref[...] * dinv_ref[...]


def _tc_dinv(degA, degB):
    return pl.pallas_call(
        _dinv_body,
        out_shape=jax.ShapeDtypeStruct(degA.shape, jnp.float32),
    )(degA, degB)


def _tc_scale(h, dinv_col):
    return pl.pallas_call(
        _scale_body,
        out_shape=jax.ShapeDtypeStruct((N, D), jnp.float32),
    )(h, dinv_col)


def _layer_body(relu, resid, want_g, h_ref, acc_ref, dinv_ref, w0_ref, w1_ref,
                b_ref, *rest):
    if resid:
        yres_ref, out_ref, *grest = rest
    else:
        out_ref, *grest = rest
    parts = []
    for p in range(NPASS):
        parts.append(acc_ref[0, p, pl.ds(0, PW), :]
                     + acc_ref[1, p, pl.ds(0, PW), :])
    t = jnp.concatenate(parts, axis=0)[:N, :] * dinv_ref[...]
    out = (jnp.dot(h_ref[...], w0_ref[...], preferred_element_type=jnp.float32)
           - jnp.dot(t, w1_ref[...], preferred_element_type=jnp.float32)
           + b_ref[...])
    if relu:
        out = jnp.maximum(out, 0.0)
    if resid:
        out = (yres_ref[...] + out) * 0.5
    out_ref[...] = out
    if want_g:
        grest[0][...] = out * dinv_ref[...]


def _tc_layer(h, acc, dinv_col, w0, w1, bk, yres=None, relu=True, want_g=True):
    out_shape = [jax.ShapeDtypeStruct((N, D), jnp.float32)]
    if want_g:
        out_shape.append(jax.ShapeDtypeStruct((N, D), jnp.float32))
    args = [h, acc, dinv_col, w0, w1, bk]
    if yres is not None:
        args.append(yres)
    res = pl.pallas_call(
        functools.partial(_layer_body, relu, yres is not None, want_g),
        out_shape=out_shape,
    )(*args)
    return res if want_g else (res[0], None)


def kernel(x, edge_index, W0, W1, b):
    src = edge_index[0]
    dst = edge_index[1]
    keep = src != dst

    dpad = ((0, 0), (0, DEPT_PAD - EPT))
    srcp = jnp.pad(jnp.where(keep, src, DUMMY).reshape(NW, EPT), dpad,
                   constant_values=DUMMY).reshape(NW, DNCHUNK, DCH)

    pid = jnp.clip(dst // PW, 0, NPASS - 1)
    pid = jnp.where(keep, pid, NPASS - 1)
    pid_s, src_s, dst_s = lax.sort((pid, src, dst), num_keys=1)
    counts = jnp.bincount(pid, length=NPASS)
    starts = jnp.concatenate([jnp.zeros((1,), counts.dtype),
                              jnp.cumsum(counts)[:-1]])
    spad = jnp.pad(src_s, (0, NW * SLOTS))
    dpad_arr = jnp.pad(dst_s, (0, NW * SLOTS))
    rank = jnp.arange(NW * SLOTS, dtype=jnp.int32)
    gsrc_l, ldst_l = [], []
    for p in range(NPASS):
        sp = lax.dynamic_slice(spad, (starts[p],), (NW * SLOTS,))
        dp = lax.dynamic_slice(dpad_arr, (starts[p],), (NW * SLOTS,))
        valid = rank < counts[p]
        inpass = valid & (sp != dp) & (jnp.clip(dp // PW, 0, NPASS - 1) == p)
        gsrc_l.append(jnp.where(valid, sp, 0))
        ldst_l.append(jnp.where(inpass, dp - p * PW, LDUMMY))
    gsrc = jnp.stack(gsrc_l).reshape(NPASS, NW, NCHP, CH).astype(jnp.int32)
    ldst = jnp.stack(ldst_l).reshape(NPASS, NW, NCHP, CH).astype(jnp.int32)

    zrows = jnp.zeros((ARPT, D), jnp.float32)
    z1d = jnp.zeros((DRPT,), jnp.float32)
    ones = jnp.ones((DCH,), jnp.float32)

    deg_parts = _sc_degree(srcp, ones, z1d)
    dinv2d = _tc_dinv(deg_parts[0].reshape(DNPAD // D, D),
                      deg_parts[1].reshape(DNPAD // D, D))
    dinv_col = dinv2d.reshape(DNPAD)[:N].reshape(N, 1)

    g = _tc_scale(x, dinv_col)

    def cheb(k, h, yres=None, relu=True, want_g=True):
        acc = _sc_segsum(g_holder[0], gsrc, ldst, zrows)
        return _tc_layer(h, acc, dinv_col, W0[k], W1[k],
                         b[k].reshape(1, D), yres=yres, relu=relu,
                         want_g=want_g)

    g_holder = [g]
    y, gy = cheb(0, x)
    g_holder[0] = gy
    for blk in range(3):
        k = 1 + 2 * blk
        h1, gh = cheb(k, y)
        g_holder[0] = gh
        y, gy = cheb(k + 1, h1, yres=y)
        g_holder[0] = gy
    y2, _ = cheb(7, y, relu=False, want_g=False)
    return (y2, y)
